# Initial kernel scaffold; baseline (speedup 1.0000x reference)
import jax
import jax.numpy as jnp
from jax.experimental import pallas as pl


def kernel(x, pos_table):
    raise NotImplementedError("write your pallas kernel here")


# speedup vs baseline: 1.6962x; 1.6962x over previous
import jax
import jax.numpy as jnp
from jax.experimental import pallas as pl


def _add_pe_kernel(x_ref, pe_ref, o_ref):
    o_ref[...] = x_ref[...] + pe_ref[...]


def kernel(x, pos_table):
    B, S, D = x.shape
    Sb = 512
    grid = (S // Sb, B)
    return pl.pallas_call(
        _add_pe_kernel,
        grid=grid,
        in_specs=[
            pl.BlockSpec((1, Sb, D), lambda s, b: (b, s, 0)),
            pl.BlockSpec((Sb, D), lambda s, b: (s, 0)),
        ],
        out_specs=pl.BlockSpec((1, Sb, D), lambda s, b: (b, s, 0)),
        out_shape=jax.ShapeDtypeStruct((B, S, D), x.dtype),
    )(x, pos_table)


# speedup vs baseline: 1.8739x; 1.1048x over previous
import jax
import jax.numpy as jnp
from jax.experimental import pallas as pl


def _add_pe_kernel(x_ref, pe_ref, o_ref):
    o_ref[...] = x_ref[...] + pe_ref[...]


def kernel(x, pos_table):
    B, S, D = x.shape
    Sb = 1024
    grid = (S // Sb, B)
    return pl.pallas_call(
        _add_pe_kernel,
        grid=grid,
        in_specs=[
            pl.BlockSpec((1, Sb, D), lambda s, b: (b, s, 0)),
            pl.BlockSpec((Sb, D), lambda s, b: (s, 0)),
        ],
        out_specs=pl.BlockSpec((1, Sb, D), lambda s, b: (b, s, 0)),
        out_shape=jax.ShapeDtypeStruct((B, S, D), x.dtype),
    )(x, pos_table)


# speedup vs baseline: 1.9969x; 1.0656x over previous
import jax
import jax.numpy as jnp
from jax.experimental import pallas as pl


def _add_pe_kernel(x_ref, pe_ref, o_ref):
    o_ref[...] = x_ref[...] + pe_ref[...]


def kernel(x, pos_table):
    B, S, D = x.shape
    Sb = 2048
    grid = (S // Sb, B)
    return pl.pallas_call(
        _add_pe_kernel,
        grid=grid,
        in_specs=[
            pl.BlockSpec((1, Sb, D), lambda s, b: (b, s, 0)),
            pl.BlockSpec((Sb, D), lambda s, b: (s, 0)),
        ],
        out_specs=pl.BlockSpec((1, Sb, D), lambda s, b: (b, s, 0)),
        out_shape=jax.ShapeDtypeStruct((B, S, D), x.dtype),
    )(x, pos_table)
